# Initial kernel scaffold; baseline (speedup 1.0000x reference)
import jax
import jax.numpy as jnp
from jax.experimental import pallas as pl


def kernel(x, edge_index, W1, b1, W2, b2, W3, b3, Wp, bp, Wf, bf, Wr, br):
    raise NotImplementedError("write your pallas kernel here")


# speedup vs baseline: 17.5811x; 17.5811x over previous
import functools

import jax
import jax.numpy as jnp
from jax import lax
from jax.experimental import pallas as pl
from jax.experimental.pallas import tpu as pltpu
from jax.experimental.pallas import tpu_sc as plsc

N = 10000
E = 320000
D = 128

NC = 2
NS = 16
NW = NC * NS
EPW = E // NW
C = 125
NCHUNK = EPW // C
NPAD = 10240
RPT = NPAD // NS

_MESH = dict(core_axis_name="c", subcore_axis_name="s")


def _worker_ids():
    c = lax.axis_index("c")
    s = lax.axis_index("s")
    return c, s, c * NS + s


@functools.partial(
    pl.kernel,
    out_type=jax.ShapeDtypeStruct((2 * NPAD, 16), jnp.float32),
    mesh=plsc.VectorSubcoreMesh(**_MESH),
    scratch_types=[
        pltpu.VMEM((NCHUNK, C), jnp.int32),
        pltpu.VMEM((C, 16), jnp.float32),
        pltpu.VMEM_SHARED((NPAD, 16), jnp.float32),
    ],
    compiler_params=pltpu.CompilerParams(use_tc_tiling_on_sc=False),
)
def _sc_degree(dst_h, ones_h, zeros_h, out_h, idx_v, ones_v, acc):
    c, s, wid = _worker_ids()
    pltpu.sync_copy(dst_h.at[wid], idx_v)
    pltpu.sync_copy(ones_h, ones_v)
    pltpu.sync_copy(zeros_h, acc.at[pl.ds(s * RPT, RPT)])
    plsc.subcore_barrier()

    def body(j, carry):
        pltpu.sync_copy(ones_v, acc.at[idx_v.at[j]], add=True)
        return carry

    lax.fori_loop(0, NCHUNK, body, 0)
    plsc.subcore_barrier()
    pltpu.sync_copy(acc.at[pl.ds(s * RPT, RPT)],
                    out_h.at[pl.ds(c * NPAD + s * RPT, RPT)])


def _make_sc_msg(d):
    @functools.partial(
        pl.kernel,
        out_type=jax.ShapeDtypeStruct((2 * NPAD, d), jnp.float32),
        mesh=plsc.VectorSubcoreMesh(**_MESH),
        scratch_types=[
            pltpu.VMEM((NCHUNK, C), jnp.int32),
            pltpu.VMEM((NCHUNK, C), jnp.int32),
            pltpu.VMEM((C, d), jnp.float32),
            pltpu.VMEM_SHARED((NPAD, d), jnp.float32),
            pltpu.SemaphoreType.DMA,
        ],
        compiler_params=pltpu.CompilerParams(use_tc_tiling_on_sc=False),
    )
    def msg(src_h, dst_h, g_h, zeros_h, out_h, src_v, dst_v, rows_v, acc, sem):
        c, s, wid = _worker_ids()
        pltpu.sync_copy(src_h.at[wid], src_v)
        pltpu.sync_copy(dst_h.at[wid], dst_v)
        pltpu.sync_copy(zeros_h, acc.at[pl.ds(s * RPT, RPT)])
        plsc.subcore_barrier()

        def body(j, carry):
            pltpu.async_copy(g_h.at[src_v.at[j]], rows_v, sem).wait()
            pltpu.sync_copy(rows_v, acc.at[dst_v.at[j]], add=True)
            return carry

        lax.fori_loop(0, NCHUNK, body, 0)
        plsc.subcore_barrier()
        pltpu.sync_copy(acc.at[pl.ds(s * RPT, RPT)],
                        out_h.at[pl.ds(c * NPAD + s * RPT, RPT)])

    return msg


_SC_MSG128 = _make_sc_msg(D)
_SC_MSG64 = _make_sc_msg(D // 2)


BLK = 400
GRID = (N // BLK,)


def _dis(degs_ref):
    deg = degs_ref[0, :, 0:1] + degs_ref[1, :, 0:1] + 1.0
    return lax.rsqrt(deg)


def _row(i):
    return (i, 0)


def _full2(i):
    return (0, 0)


def _parts(i):
    return (0, i, 0)


_SPEC_DEGS = pl.BlockSpec((2, BLK, 16), _parts)


def _tc_stage1_body(x_ref, xt_ref, w1_ref, wp_ref, bp_ref, degs_ref,
                    g1_ref, ppr_ref):
    dis = _dis(degs_ref)
    g1_ref[...] = dis * jnp.dot(x_ref[...], w1_ref[...],
                                preferred_element_type=jnp.float32)
    ppr_ref[...] = jax.nn.relu(
        jnp.dot(xt_ref[...], wp_ref[...], preferred_element_type=jnp.float32)
        + bp_ref[...])


def _tc_stage1(x, xt, w1, wp, bp, degs):
    return pl.pallas_call(
        _tc_stage1_body,
        grid=GRID,
        in_specs=[
            pl.BlockSpec((BLK, D), _row),
            pl.BlockSpec((BLK, 10), _row),
            pl.BlockSpec((D, D), _full2),
            pl.BlockSpec((10, 32), _full2),
            pl.BlockSpec((1, 32), _full2),
            _SPEC_DEGS,
        ],
        out_specs=[
            pl.BlockSpec((BLK, D), _row),
            pl.BlockSpec((BLK, 32), _row),
        ],
        out_shape=[
            jax.ShapeDtypeStruct((N, D), jnp.float32),
            jax.ShapeDtypeStruct((N, 32), jnp.float32),
        ],
    )(x, xt, w1, wp, bp, degs)


def _tc_combine_body(p_ref, g_ref, b_ref, w_ref, degs_ref, out_ref):
    dis = _dis(degs_ref)
    a = jax.nn.relu(dis * (p_ref[0] + p_ref[1] + g_ref[...]) + b_ref[...])
    out_ref[...] = dis * jnp.dot(a, w_ref[...],
                                 preferred_element_type=jnp.float32)


def _tc_combine(p, g, b, w, degs):
    din = g.shape[1]
    dout = w.shape[1]
    return pl.pallas_call(
        _tc_combine_body,
        grid=GRID,
        in_specs=[
            pl.BlockSpec((2, BLK, din), _parts),
            pl.BlockSpec((BLK, din), _row),
            pl.BlockSpec((1, din), _full2),
            pl.BlockSpec((din, dout), _full2),
            _SPEC_DEGS,
        ],
        out_specs=pl.BlockSpec((BLK, dout), _row),
        out_shape=jax.ShapeDtypeStruct((N, dout), jnp.float32),
    )(p, g, b, w, degs)


def _tc_head_body(p_ref, g_ref, b3_ref, ppr_ref, wf1_ref, wf2_ref, bf_ref,
                  wr_ref, br_ref, degs_ref, out_ref):
    dis = _dis(degs_ref)
    h3 = jax.nn.relu(dis * (p_ref[0] + p_ref[1] + g_ref[...]) + b3_ref[...])
    fused = jax.nn.relu(
        jnp.dot(h3, wf1_ref[...], preferred_element_type=jnp.float32)
        + jnp.dot(ppr_ref[...], wf2_ref[...], preferred_element_type=jnp.float32)
        + bf_ref[...])
    out_ref[...] = jax.nn.sigmoid(
        jnp.dot(fused, wr_ref[...], preferred_element_type=jnp.float32)
        + br_ref[...])


def _tc_head(p, g, b3, ppr, wf1, wf2, bf, wr, br, degs):
    h = D // 2
    return pl.pallas_call(
        _tc_head_body,
        grid=GRID,
        in_specs=[
            pl.BlockSpec((2, BLK, h), _parts),
            pl.BlockSpec((BLK, h), _row),
            pl.BlockSpec((1, h), _full2),
            pl.BlockSpec((BLK, 32), _row),
            pl.BlockSpec((h, h), _full2),
            pl.BlockSpec((32, h), _full2),
            pl.BlockSpec((1, h), _full2),
            pl.BlockSpec((h, 1), _full2),
            pl.BlockSpec((1, 1), _full2),
            _SPEC_DEGS,
        ],
        out_specs=pl.BlockSpec((BLK, 1), _row),
        out_shape=jax.ShapeDtypeStruct((N, 1), jnp.float32),
    )(p, g, b3, ppr, wf1, wf2, bf, wr, br, degs)


def kernel(x, edge_index, W1, b1, W2, b2, W3, b3, Wp, bp, Wf, bf, Wr, br):
    src3 = edge_index[0].reshape(NW, NCHUNK, C)
    dst3 = edge_index[1].reshape(NW, NCHUNK, C)
    xt = x[:, D - 10:]
    ones16 = jnp.ones((C, 16), jnp.float32)
    zeros16 = jnp.zeros((RPT, 16), jnp.float32)
    zeros128 = jnp.zeros((RPT, D), jnp.float32)
    zeros64 = jnp.zeros((RPT, D // 2), jnp.float32)

    degs = _sc_degree(dst3, ones16, zeros16).reshape(2, NPAD, 16)

    g1, ppr = _tc_stage1(x, xt, W1, Wp, bp.reshape(1, 32), degs)
    p1 = _SC_MSG128(src3, dst3, g1, zeros128).reshape(2, NPAD, D)
    g2 = _tc_combine(p1, g1, b1.reshape(1, D), W2, degs)
    p2 = _SC_MSG128(src3, dst3, g2, zeros128).reshape(2, NPAD, D)
    g3 = _tc_combine(p2, g2, b2.reshape(1, D), W3, degs)
    p3 = _SC_MSG64(src3, dst3, g3, zeros64).reshape(2, NPAD, D // 2)

    out = _tc_head(p3, g3, b3.reshape(1, D // 2), ppr,
                   Wf[:D // 2], Wf[D // 2:], bf.reshape(1, D // 2),
                   Wr, br.reshape(1, 1), degs)
    return out.reshape(N)
